# baseline (speedup 1.0000x reference)
import functools

import jax
import jax.numpy as jnp
from jax import lax
from jax.experimental import pallas as pl
from jax.experimental.pallas import tpu as pltpu
from jax.experimental.pallas import tpu_sc as plsc

D = 32
NC = 2
NS = 16
NW = NC * NS
G = 128


@functools.cache
def _build(S: int, K: int, V: int):
    KB = K // NW
    NGS = KB // G
    NKG = KB // 128
    mesh = plsc.VectorSubcoreMesh(core_axis_name="c", subcore_axis_name="s")

    @functools.partial(
        pl.kernel,
        mesh=mesh,
        compiler_params=pltpu.CompilerParams(use_tc_tiling_on_sc=False),
        out_type=jax.ShapeDtypeStruct((S, D // 8, K // 128, 8, 128),
                                      jnp.float32),
        scratch_types=[
            pltpu.VMEM((S, KB), jnp.int32),
            pltpu.VMEM((NGS, G, D), jnp.float32),
            pltpu.VMEM((NGS, G, D), jnp.float32),
            pltpu.SemaphoreType.DMA,
            pltpu.SemaphoreType.DMA,
            pltpu.SemaphoreType.DMA,
            pltpu.SemaphoreType.DMA,
        ],
    )
    def emb_kernel(idx_hbm, tab_hbm, out_hbm, sidx, rows0, rows1,
                   sg0, sg1, so0, so1):
        wid = lax.axis_index("s") * NC + lax.axis_index("c")

        pltpu.sync_copy(idx_hbm.at[:, wid], sidx)

        def fire_gathers(s, rows, sem):
            for g in range(NGS):
                pltpu.async_copy(
                    tab_hbm.at[sidx.at[s].at[pl.ds(g * G, G)]],
                    rows.at[g], sem)

        def wait_gathers(rows, sem):
            pltpu.make_async_copy(tab_hbm.at[pl.ds(0, G)], rows.at[0],
                                  sem).wait()
            pltpu.make_async_copy(tab_hbm.at[pl.ds(0, G)], rows.at[1],
                                  sem).wait()
            pltpu.make_async_copy(tab_hbm.at[pl.ds(0, G)], rows.at[2],
                                  sem).wait()
            pltpu.make_async_copy(tab_hbm.at[pl.ds(0, G)], rows.at[3],
                                  sem).wait()

        def fire_out(s, rows, sem):
            for j in range(D):
                pltpu.async_copy(
                    rows.at[:, :, j],
                    out_hbm.at[s, j // 8, pl.ds(wid * NKG, NKG), j % 8],
                    sem)

        def wait_out(rows, sem):
            for j in range(D):
                pltpu.make_async_copy(
                    rows.at[:, :, j],
                    out_hbm.at[0, j // 8, pl.ds(0, NKG), j % 8], sem).wait()

        fire_gathers(0, rows0, sg0)
        fire_gathers(1, rows1, sg1)

        def s_body(h, carry):
            s0 = 2 * h
            s1 = s0 + 1
            wait_gathers(rows0, sg0)
            fire_out(s0, rows0, so0)
            wait_gathers(rows1, sg1)
            fire_out(s1, rows1, so1)

            @pl.when(s0 + 2 < S)
            def _():
                wait_out(rows0, so0)
                fire_gathers(s0 + 2, rows0, sg0)

            @pl.when(s1 + 2 < S)
            def _():
                wait_out(rows1, so1)
                fire_gathers(s1 + 2, rows1, sg1)
            return carry

        lax.fori_loop(0, S // 2, s_body, 0)
        wait_out(rows0, so0)
        wait_out(rows1, so1)

    return emb_kernel


def kernel(token_ids, weights):
    K, S = token_ids.shape
    V = weights.shape[0]
    idx = token_ids.T.reshape(S, NW, K // NW).astype(jnp.int32)
    out5 = _build(S, K, V)(idx, weights)
    return out5.transpose(2, 4, 0, 1, 3).reshape(K, S, D)


# speedup vs baseline: 55.4288x; 55.4288x over previous
import functools

import jax
import jax.numpy as jnp
from jax import lax
from jax.experimental import pallas as pl
from jax.experimental.pallas import tpu as pltpu
from jax.experimental.pallas import tpu_sc as plsc

D = 32
NC = 2
NS = 16
NW = NC * NS
G = 128


@functools.cache
def _build(S: int, K: int, V: int):
    KB = K // NW
    NGS = KB // G
    mesh = plsc.VectorSubcoreMesh(core_axis_name="c", subcore_axis_name="s")

    @functools.partial(
        pl.kernel,
        mesh=mesh,
        compiler_params=pltpu.CompilerParams(use_tc_tiling_on_sc=False),
        out_type=jax.ShapeDtypeStruct((K, S, D), jnp.float32),
        scratch_types=[
            pltpu.VMEM((S, KB), jnp.int32),
            pltpu.VMEM((KB, D), jnp.float32),
            pltpu.VMEM((KB, D), jnp.float32),
            pltpu.SemaphoreType.DMA,
            pltpu.SemaphoreType.DMA,
            pltpu.SemaphoreType.DMA,
            pltpu.SemaphoreType.DMA,
        ],
    )
    def emb_kernel(idx_hbm, tab_hbm, out_hbm, sidx, rows0, rows1,
                   sg0, sg1, so0, so1):
        wid = lax.axis_index("s") * NC + lax.axis_index("c")

        pltpu.sync_copy(idx_hbm.at[:, wid], sidx)

        def fire_gathers(s, rows, sem):
            for g in range(NGS):
                pltpu.async_copy(
                    tab_hbm.at[sidx.at[s].at[pl.ds(g * G, G)]],
                    rows.at[pl.ds(g * G, G)], sem)

        def wait_gathers(rows, sem):
            pltpu.make_async_copy(tab_hbm.at[pl.ds(0, KB)], rows, sem).wait()

        def fire_out(s, rows, sem):
            pltpu.async_copy(rows, out_hbm.at[pl.ds(wid * KB, KB), s], sem)

        def wait_out(rows, sem):
            pltpu.make_async_copy(rows, out_hbm.at[pl.ds(0, KB), 0],
                                  sem).wait()

        fire_gathers(0, rows0, sg0)
        fire_gathers(1, rows1, sg1)

        def s_body(h, carry):
            s0 = 2 * h
            s1 = s0 + 1
            wait_gathers(rows0, sg0)
            fire_out(s0, rows0, so0)
            wait_gathers(rows1, sg1)
            fire_out(s1, rows1, so1)

            @pl.when(s0 + 2 < S)
            def _():
                wait_out(rows0, so0)
                fire_gathers(s0 + 2, rows0, sg0)

            @pl.when(s1 + 2 < S)
            def _():
                wait_out(rows1, so1)
                fire_gathers(s1 + 2, rows1, sg1)
            return carry

        lax.fori_loop(0, S // 2, s_body, 0)
        wait_out(rows0, so0)
        wait_out(rows1, so1)

    return emb_kernel


def kernel(token_ids, weights):
    K, S = token_ids.shape
    V = weights.shape[0]
    idx = token_ids.T.reshape(S, NW, K // NW).astype(jnp.int32)
    return _build(S, K, V)(idx, weights)
